# baseline (speedup 1.0000x reference)
import functools
from math import sqrt

import jax
import jax.numpy as jnp
from jax import lax
from jax.experimental import pallas as pl
from jax.experimental.pallas import tpu as pltpu
from jax.experimental.pallas import tpu_sc as plsc

_MODEL_DIM = 64
_SCALE = sqrt(_MODEL_DIM)


def _make_sc_lookup(vocab, dim, n_tokens):
    info = plsc.get_sparse_core_info()
    nc, ns, lanes = info.num_cores, info.num_subcores, info.num_lanes
    nw = nc * ns
    assert n_tokens % nw == 0
    per_w = n_tokens // nw
    chunk = 512
    while per_w % chunk:
        chunk //= 2
    n_chunks = per_w // chunk
    mesh = plsc.VectorSubcoreMesh(core_axis_name="c", subcore_axis_name="s")

    @functools.partial(
        pl.kernel,
        mesh=mesh,
        compiler_params=pltpu.CompilerParams(use_tc_tiling_on_sc=False),
        out_type=jax.ShapeDtypeStruct((n_tokens, dim), jnp.float32),
        scratch_types=[
            pltpu.VMEM((chunk,), jnp.int32),
            pltpu.VMEM((chunk, dim), jnp.float32),
            pltpu.SemaphoreType.DMA,
        ],
    )
    def k(idx_hbm, table_hbm, out_hbm, idx_v, rows_v, sem):
        wid = lax.axis_index("s") * nc + lax.axis_index("c")
        base = wid * per_w

        def body(i, carry):
            off = base + i * chunk
            pltpu.sync_copy(idx_hbm.at[pl.ds(off, chunk)], idx_v)
            pltpu.async_copy(table_hbm.at[idx_v], rows_v, sem).wait()

            def srow(r, c2):
                for q in range(dim // lanes):
                    sl = pl.ds(q * lanes, lanes)
                    rows_v[r, sl] = rows_v[r, sl] * _SCALE
                return c2

            lax.fori_loop(0, chunk, srow, 0, unroll=2)
            pltpu.sync_copy(rows_v, out_hbm.at[pl.ds(off, chunk)])
            return carry

        lax.fori_loop(0, n_chunks, body, 0)

    return k


def kernel(inputs, weight):
    b, t = inputs.shape
    vocab, dim = weight.shape
    idx = inputs.reshape(-1).astype(jnp.int32)
    lookup = _make_sc_lookup(vocab, dim, b * t)
    out = lookup(idx, weight)
    return out.reshape(b, t, dim)


# speedup vs baseline: 1.0908x; 1.0908x over previous
import functools
from math import sqrt

import jax
import jax.numpy as jnp
from jax import lax
from jax.experimental import pallas as pl
from jax.experimental.pallas import tpu as pltpu
from jax.experimental.pallas import tpu_sc as plsc

_MODEL_DIM = 64
_SCALE = sqrt(_MODEL_DIM)


def _make_sc_lookup(vocab, dim, n_tokens):
    info = plsc.get_sparse_core_info()
    nc, ns, lanes = info.num_cores, info.num_subcores, info.num_lanes
    nw = nc * ns
    assert n_tokens % nw == 0
    per_w = n_tokens // nw
    chunk = 512
    while per_w % (2 * chunk):
        chunk //= 2
    n_chunks = per_w // chunk
    mesh = plsc.VectorSubcoreMesh(core_axis_name="c", subcore_axis_name="s")

    @functools.partial(
        pl.kernel,
        mesh=mesh,
        compiler_params=pltpu.CompilerParams(use_tc_tiling_on_sc=False),
        out_type=jax.ShapeDtypeStruct((n_tokens, dim), jnp.float32),
        scratch_types=[
            pltpu.VMEM((n_chunks, chunk), jnp.int32),
            pltpu.VMEM((chunk, dim), jnp.float32),
            pltpu.VMEM((chunk, dim), jnp.float32),
            pltpu.SemaphoreType.DMA,
            pltpu.SemaphoreType.DMA,
            pltpu.SemaphoreType.DMA,
            pltpu.SemaphoreType.DMA,
        ],
    )
    def k(idx_hbm, table_hbm, out_hbm, idx_v, rows0, rows1, g0, g1, o0, o1):
        wid = lax.axis_index("s") * nc + lax.axis_index("c")
        base = wid * per_w
        rows = (rows0, rows1)
        gsem = (g0, g1)
        osem = (o0, o1)

        pltpu.sync_copy(
            idx_hbm.at[pl.ds(wid * n_chunks, n_chunks)], idx_v
        )

        def gather(i, b):
            pltpu.async_copy(table_hbm.at[idx_v.at[i]], rows[b], gsem[b])

        def gather_wait(b):
            pltpu.make_async_copy(table_hbm.at[idx_v.at[0]], rows[b], gsem[b]).wait()

        def out_start(i, b):
            pltpu.async_copy(rows[b], out_hbm.at[pl.ds(base + i * chunk, chunk)], osem[b])

        def out_wait(b):
            pltpu.make_async_copy(
                rows[b], out_hbm.at[pl.ds(base, chunk)], osem[b]
            ).wait()

        def scale(b):
            r = rows[b]

            def srow(row, c2):
                for q in range(dim // lanes):
                    sl = pl.ds(q * lanes, lanes)
                    r[row, sl] = r[row, sl] * _SCALE
                return c2

            lax.fori_loop(0, chunk, srow, 0, unroll=4)

        gather(0, 0)

        def body(g, carry):
            i0 = g * 2
            @pl.when(i0 > 0)
            def _():
                out_wait(1)

            gather(i0 + 1, 1)
            gather_wait(0)
            scale(0)
            out_start(i0, 0)
            out_wait(0)

            @pl.when(i0 + 2 < n_chunks)
            def _():
                gather(i0 + 2, 0)

            gather_wait(1)
            scale(1)
            out_start(i0 + 1, 1)
            return carry

        lax.fori_loop(0, n_chunks // 2, body, 0)
        out_wait(1)

    return k


def kernel(inputs, weight):
    b, t = inputs.shape
    vocab, dim = weight.shape
    n_tokens = b * t
    lookup = _make_sc_lookup(vocab, dim, n_tokens)
    info = plsc.get_sparse_core_info()
    nw = info.num_cores * info.num_subcores
    per_w = n_tokens // nw
    chunk = 512
    while per_w % (2 * chunk):
        chunk //= 2
    idx = inputs.reshape(n_tokens // chunk, chunk).astype(jnp.int32)
    out = lookup(idx, weight)
    return out.reshape(b, t, dim)
